# baseline (speedup 1.0000x reference)
import functools

import jax
import jax.numpy as jnp
from jax.experimental import pallas as pl
from jax.experimental.pallas import tpu as pltpu

BN_EPS = 1e-5
_INTERPRET = False


def _msg_kernel(xj_ref, e_ref, w1a_ref, w1b_ref, b1_ref, out_ref):
    xj = xj_ref[...]
    eb = e_ref[...]
    z = (jnp.dot(xj, w1a_ref[...], preferred_element_type=jnp.float32)
         + jnp.dot(eb, w1b_ref[...], preferred_element_type=jnp.float32)
         + b1_ref[...])
    out_ref[...] = jnp.maximum(z, 0.0)


def _lin_stats_kernel(a_ref, b_ref, c_ref, w_ref, bias_ref, z_ref, acc_ref):
    h = a_ref[...] + b_ref[...] + c_ref[...]
    z = jnp.dot(h, w_ref[...], preferred_element_type=jnp.float32) + bias_ref[...]
    z_ref[...] = z
    @pl.when(pl.program_id(0) == 0)
    def _():
        acc_ref[...] = jnp.zeros_like(acc_ref)
    acc_ref[0, :] += jnp.sum(z, axis=0)
    acc_ref[1, :] += jnp.sum(z * z, axis=0)


def _bn_lin_stats_kernel(z_ref, s_ref, o_ref, w_ref, bias_ref, z2_ref, acc_ref):
    u = jnp.maximum(z_ref[...] * s_ref[...] + o_ref[...], 0.0)
    z2 = jnp.dot(u, w_ref[...], preferred_element_type=jnp.float32) + bias_ref[...]
    z2_ref[...] = z2
    @pl.when(pl.program_id(0) == 0)
    def _():
        acc_ref[...] = jnp.zeros_like(acc_ref)
    acc_ref[0, :] += jnp.sum(z2, axis=0)
    acc_ref[1, :] += jnp.sum(z2 * z2, axis=0)


def _bn_relu_kernel(z_ref, s_ref, o_ref, out_ref):
    out_ref[...] = jnp.maximum(z_ref[...] * s_ref[...] + o_ref[...], 0.0)


def _scale_offset(acc, count, gamma, beta):
    mu = acc[0] / count
    var = acc[1] / count - mu * mu
    s = gamma * jax.lax.rsqrt(var + BN_EPS)
    return s, beta - mu * s


def _blk(shape, imap):
    return pl.BlockSpec(shape, imap)


def kernel(x, e, edge_index, W1, b1, W2, b2, g2, be2, W3, b3, g3, be3,
           W4, b4, g4, be4, W5, b5, g5, be5):
    N, D = x.shape
    E = e.shape[0]
    H = W2.shape[1]
    j = edge_index[0]
    i = edge_index[1]
    x_j = jnp.take(x, j, axis=0)
    x_i = jnp.take(x, i, axis=0)

    BE = 8000
    GE = E // BE
    row = lambda idx: (idx, 0)
    zero2 = lambda idx: (0, 0)

    msg = pl.pallas_call(
        _msg_kernel,
        grid=(GE,),
        in_specs=[_blk((BE, D), row), _blk((BE, D), row),
                  _blk((D, D), zero2), _blk((D, D), zero2),
                  pl.BlockSpec((1, D), zero2)],
        out_specs=_blk((BE, D), row),
        out_shape=jax.ShapeDtypeStruct((E, D), jnp.float32),
        interpret=_INTERPRET,
    )(x_j, e, W1[:D], W1[D:], b1.reshape(1, D))

    n_agg = jax.ops.segment_sum(msg, i, num_segments=N)

    BN = 10000
    GN = N // BN
    z2, acc2 = pl.pallas_call(
        _lin_stats_kernel,
        grid=(GN,),
        in_specs=[_blk((BN, D), row), _blk((BN, D), row), _blk((BN, D), row),
                  _blk((D, H), zero2), pl.BlockSpec((1, H), zero2)],
        out_specs=[_blk((BN, H), row), _blk((2, H), zero2)],
        out_shape=[jax.ShapeDtypeStruct((N, H), jnp.float32),
                   jax.ShapeDtypeStruct((2, H), jnp.float32)],
        interpret=_INTERPRET,
    )(n_agg, x, jnp.zeros_like(x), W2, b2.reshape(1, H))
    s2, o2 = _scale_offset(acc2, N, g2, be2)

    z3, acc3 = pl.pallas_call(
        _bn_lin_stats_kernel,
        grid=(GN,),
        in_specs=[_blk((BN, H), row), pl.BlockSpec((1, H), zero2),
                  pl.BlockSpec((1, H), zero2), _blk((H, H), zero2),
                  pl.BlockSpec((1, H), zero2)],
        out_specs=[_blk((BN, H), row), _blk((2, H), zero2)],
        out_shape=[jax.ShapeDtypeStruct((N, H), jnp.float32),
                   jax.ShapeDtypeStruct((2, H), jnp.float32)],
        interpret=_INTERPRET,
    )(z2, s2.reshape(1, H), o2.reshape(1, H), W3, b3.reshape(1, H))
    s3, o3 = _scale_offset(acc3, N, g3, be3)

    n_out = pl.pallas_call(
        _bn_relu_kernel,
        grid=(GN,),
        in_specs=[_blk((BN, H), row), pl.BlockSpec((1, H), zero2),
                  pl.BlockSpec((1, H), zero2)],
        out_specs=_blk((BN, H), row),
        out_shape=jax.ShapeDtypeStruct((N, H), jnp.float32),
        interpret=_INTERPRET,
    )(z3, s3.reshape(1, H), o3.reshape(1, H))

    z4, acc4 = pl.pallas_call(
        _lin_stats_kernel,
        grid=(GE,),
        in_specs=[_blk((BE, D), row), _blk((BE, D), row), _blk((BE, D), row),
                  _blk((D, H), zero2), pl.BlockSpec((1, H), zero2)],
        out_specs=[_blk((BE, H), row), _blk((2, H), zero2)],
        out_shape=[jax.ShapeDtypeStruct((E, H), jnp.float32),
                   jax.ShapeDtypeStruct((2, H), jnp.float32)],
        interpret=_INTERPRET,
    )(x_i, x_j, e, W4, b4.reshape(1, H))
    s4, o4 = _scale_offset(acc4, E, g4, be4)

    z5, acc5 = pl.pallas_call(
        _bn_lin_stats_kernel,
        grid=(GE,),
        in_specs=[_blk((BE, H), row), pl.BlockSpec((1, H), zero2),
                  pl.BlockSpec((1, H), zero2), _blk((H, H), zero2),
                  pl.BlockSpec((1, H), zero2)],
        out_specs=[_blk((BE, H), row), _blk((2, H), zero2)],
        out_shape=[jax.ShapeDtypeStruct((E, H), jnp.float32),
                   jax.ShapeDtypeStruct((2, H), jnp.float32)],
        interpret=_INTERPRET,
    )(z4, s4.reshape(1, H), o4.reshape(1, H), W5, b5.reshape(1, H))
    s5, o5 = _scale_offset(acc5, E, g5, be5)

    e_out = pl.pallas_call(
        _bn_relu_kernel,
        grid=(GE,),
        in_specs=[_blk((BE, H), row), pl.BlockSpec((1, H), zero2),
                  pl.BlockSpec((1, H), zero2)],
        out_specs=_blk((BE, H), row),
        out_shape=jax.ShapeDtypeStruct((E, H), jnp.float32),
        interpret=_INTERPRET,
    )(z5, s5.reshape(1, H), o5.reshape(1, H))

    return (n_out, e_out)


# speedup vs baseline: 1.3250x; 1.3250x over previous
import functools

import jax
import jax.numpy as jnp
from jax import lax
from jax.experimental import pallas as pl
from jax.experimental.pallas import tpu as pltpu
from jax.experimental.pallas import tpu_sc as plsc

BN_EPS = 1e-5

_NC, _NS, _L = 2, 16, 16
_CH = 80
_TW = 128
_C2 = 80
_S2 = _C2


def _tables_kernel(x_ref, wt0_ref, wt1_ref, bt0_ref, bt1_ref, tj_ref):
    xb = x_ref[...]
    tj0 = jnp.dot(xb, wt0_ref[...], preferred_element_type=jnp.float32) + bt0_ref[...]
    tj1 = jnp.dot(xb, wt1_ref[...], preferred_element_type=jnp.float32) + bt1_ref[...]
    pad = jnp.zeros((xb.shape[0], _TW - 32), jnp.float32)
    tj_ref[0] = jnp.concatenate([tj0, pad], axis=1)
    tj_ref[1] = jnp.concatenate([tj1, pad], axis=1)


def _ew_kernel(e_ref, wc0_ref, wc1_ref, bc0_ref, bc1_ref, ew_ref):
    eb = e_ref[...]
    ew_ref[0] = jnp.dot(eb, wc0_ref[...], preferred_element_type=jnp.float32) + bc0_ref[...]
    ew_ref[1] = jnp.dot(eb, wc1_ref[...], preferred_element_type=jnp.float32) + bc1_ref[...]


def _node1_kernel(na_ref, nb_ref, x_ref, w2a_ref, w2b_ref, b2_ref, z_ref, acc_ref):
    xb = x_ref[...]
    ha = na_ref[...] + xb[:, :16]
    hb = nb_ref[...] + xb[:, 16:]
    z = (jnp.dot(ha, w2a_ref[...], preferred_element_type=jnp.float32)
         + jnp.dot(hb, w2b_ref[...], preferred_element_type=jnp.float32)
         + b2_ref[...])
    z_ref[...] = z
    @pl.when(pl.program_id(0) == 0)
    def _():
        acc_ref[...] = jnp.zeros_like(acc_ref)
    acc_ref[0, :] += jnp.sum(z, axis=0)
    acc_ref[1, :] += jnp.sum(z * z, axis=0)


def _bn_lin_stats_kernel(z_ref, s_ref, o_ref, w_ref, bias_ref, z2_ref, acc_ref):
    u = jnp.maximum(z_ref[...] * s_ref[...] + o_ref[...], 0.0)
    z2 = jnp.dot(u, w_ref[...], preferred_element_type=jnp.float32) + bias_ref[...]
    z2_ref[...] = z2
    @pl.when(pl.program_id(0) == 0)
    def _():
        acc_ref[...] = jnp.zeros_like(acc_ref)
    acc_ref[0, :] += jnp.sum(z2, axis=0)
    acc_ref[1, :] += jnp.sum(z2 * z2, axis=0)


def _bn_relu_kernel(z_ref, s_ref, o_ref, out_ref):
    out_ref[...] = jnp.maximum(z_ref[...] * s_ref[...] + o_ref[...], 0.0)


def _edge_stats_kernel(za_ref, zb_ref, s4_ref, o4_ref, w5a_ref, w5b_ref,
                       b5_ref, acc_ref):
    zna = jnp.maximum(za_ref[...] * s4_ref[:, :16] + o4_ref[:, :16], 0.0)
    znb = jnp.maximum(zb_ref[...] * s4_ref[:, 16:] + o4_ref[:, 16:], 0.0)
    z5 = (jnp.dot(zna, w5a_ref[...], preferred_element_type=jnp.float32)
          + jnp.dot(znb, w5b_ref[...], preferred_element_type=jnp.float32)
          + b5_ref[...])
    @pl.when(pl.program_id(0) == 0)
    def _():
        acc_ref[...] = jnp.zeros_like(acc_ref)
    acc_ref[0, :] += jnp.sum(z5, axis=0)
    acc_ref[1, :] += jnp.sum(z5 * z5, axis=0)


def _edge_out_kernel(za_ref, zb_ref, s4_ref, o4_ref, w5a_ref, w5b_ref,
                     b5_ref, s5_ref, o5_ref, out_ref):
    zna = jnp.maximum(za_ref[...] * s4_ref[:, :16] + o4_ref[:, :16], 0.0)
    znb = jnp.maximum(zb_ref[...] * s4_ref[:, 16:] + o4_ref[:, 16:], 0.0)
    z5 = (jnp.dot(zna, w5a_ref[...], preferred_element_type=jnp.float32)
          + jnp.dot(znb, w5b_ref[...], preferred_element_type=jnp.float32)
          + b5_ref[...])
    out_ref[...] = jnp.maximum(z5 * s5_ref[...] + o5_ref[...], 0.0)


def _sc1_body(E, tj_hbm, ew_hbm, jc_hbm, ic_hbm,
              msga_hbm, msgb_hbm, z4a_hbm, z4b_hbm, stats_hbm,
              jbuf, ibuf, gbuf, tibuf, ewbuf, mbuf, z4buf, sbuf,
              isem0, isem1, gsem0, gsem1, wsem0, wsem1):
    ept = E // _NS
    c = lax.axis_index("c")
    s = lax.axis_index("s")
    base0 = s * ept
    isems = (isem0, isem1)
    gsems = (gsem0, gsem1)
    wsems = (wsem0, wsem1)
    K = ept // _CH

    def bof(k):
        return pl.ds(pl.multiple_of(base0 + k * _CH, 8), _CH)

    def fire_idx(k, b):
        off = pl.multiple_of(c * E + base0 + k * _CH, 8)
        pltpu.async_copy(jc_hbm.at[pl.ds(off, _CH)], jbuf.at[b], isems[b])
        pltpu.async_copy(ic_hbm.at[pl.ds(off, _CH)], ibuf.at[b], isems[b])

    def wait_idx(b):
        pltpu.make_async_copy(jc_hbm.at[pl.ds(0, _CH)], jbuf.at[b], isems[b]).wait()
        pltpu.make_async_copy(ic_hbm.at[pl.ds(0, _CH)], ibuf.at[b], isems[b]).wait()

    def fire_gather(k, b):
        pltpu.async_copy(tj_hbm.at[jbuf.at[b]], gbuf.at[b], gsems[b])
        pltpu.async_copy(tj_hbm.at[ibuf.at[b]], tibuf.at[b], gsems[b])
        off = pl.multiple_of(c * E + base0 + k * _CH, 8)
        pltpu.async_copy(ew_hbm.at[pl.ds(off, _CH)], ewbuf.at[b], gsems[b])

    def wait_gather(b):
        pltpu.make_async_copy(tj_hbm.at[pl.ds(0, _CH)], gbuf.at[b], gsems[b]).wait()
        pltpu.make_async_copy(tj_hbm.at[pl.ds(0, _CH)], tibuf.at[b], gsems[b]).wait()
        pltpu.make_async_copy(ew_hbm.at[pl.ds(0, _CH)], ewbuf.at[b], gsems[b]).wait()

    def fire_write(k, b):
        @pl.when(c == 0)
        def _():
            pltpu.async_copy(mbuf.at[b], msga_hbm.at[bof(k)], wsems[b])
            pltpu.async_copy(z4buf.at[b], z4a_hbm.at[bof(k)], wsems[b])

        @pl.when(c == 1)
        def _():
            pltpu.async_copy(mbuf.at[b], msgb_hbm.at[bof(k)], wsems[b])
            pltpu.async_copy(z4buf.at[b], z4b_hbm.at[bof(k)], wsems[b])

    def wait_write(b):
        pltpu.make_async_copy(mbuf.at[b], msga_hbm.at[pl.ds(0, _CH)], wsems[b]).wait()
        pltpu.make_async_copy(z4buf.at[b], z4a_hbm.at[pl.ds(0, _CH)], wsems[b]).wait()

    fire_idx(0, 0)
    wait_idx(0)
    fire_gather(0, 0)
    fire_idx(1, 1)

    def pair(q, carry):
        tsum, tsq = carry
        for b in range(2):
            k = 2 * q + b

            @pl.when(k + 1 < K)
            def _():
                wait_idx(1 - b)
                fire_gather(k + 1, 1 - b)

            wait_gather(b)

            @pl.when(k + 2 < K)
            def _():
                fire_idx(k + 2, b)

            @pl.when(k >= 2)
            def _():
                wait_write(b)

            def edge(r, cr):
                cs, cq = cr
                g1 = gbuf[b, r, pl.ds(0, 16)]
                g2 = gbuf[b, r, pl.ds(16, 16)]
                tv = tibuf[b, r, pl.ds(16, 16)]
                w1 = ewbuf[b, r, pl.ds(0, 16)]
                w2 = ewbuf[b, r, pl.ds(16, 16)]
                mbuf[b, r] = jnp.maximum(g1 + w1, 0.0)
                z = g2 + tv + w2
                z4buf[b, r] = z
                return (cs + z, cq + z * z)

            zero16 = jnp.zeros((16,), jnp.float32)
            csum, csq = lax.fori_loop(0, _CH, edge, (zero16, zero16))
            tsum = tsum + csum
            tsq = tsq + csq
            fire_write(k, b)
        return (tsum, tsq)

    zero16 = jnp.zeros((16,), jnp.float32)
    tsum, tsq = lax.fori_loop(0, K // 2, pair, (zero16, zero16))

    wait_write(0)
    wait_write(1)

    sbuf[0] = tsum
    sbuf[1] = tsq
    pltpu.sync_copy(sbuf, stats_hbm.at[c * _NS + s])


def _sc2_body(Npad, E, do_scatter, msga_hbm, msgb_hbm, ivr_hbm,
              nagga_hbm, naggb_hbm, accum, mbuf, i2buf, psem0, psem1):
    ept = E // _NS
    npt = Npad // _NS
    c = lax.axis_index("c")
    s = lax.axis_index("s")
    msems = (psem0, psem1)
    K = ept // _C2

    def zb(r, carry):
        mbuf[0, r] = jnp.zeros((16,), jnp.float32)
        return carry
    lax.fori_loop(0, _C2, zb, 0)
    for t in range(npt // _C2):
        pltpu.sync_copy(mbuf.at[0],
                        accum.at[pl.ds(pl.multiple_of(s * npt + t * _C2, 8), _C2)])
    plsc.subcore_barrier()

    base0 = s * ept

    def fire(k, b):
        base = pl.multiple_of(base0 + k * _C2, 8)

        @pl.when(c == 0)
        def _():
            pltpu.async_copy(msga_hbm.at[pl.ds(base, _C2)], mbuf.at[b], msems[b])

        @pl.when(c == 1)
        def _():
            pltpu.async_copy(msgb_hbm.at[pl.ds(base, _C2)], mbuf.at[b], msems[b])

        pltpu.async_copy(ivr_hbm.at[pl.ds(base // _S2, 1)],
                         i2buf.at[pl.ds(b, 1)], msems[b])

    def wait(b):
        pltpu.make_async_copy(msga_hbm.at[pl.ds(0, _C2)], mbuf.at[b],
                              msems[b]).wait()
        pltpu.make_async_copy(ivr_hbm.at[pl.ds(0, 1)], i2buf.at[pl.ds(b, 1)],
                              msems[b]).wait()

    if do_scatter:
        fire(0, 0)

        def pair(q, carry):
            for b in range(2):
                k = 2 * q + b

                @pl.when(k + 1 < K)
                def _():
                    fire(k + 1, 1 - b)

                wait(b)
                pltpu.sync_copy(mbuf.at[b], accum.at[i2buf.at[b]], add=True)
            return carry

        lax.fori_loop(0, K // 2, pair, 0)

        plsc.subcore_barrier()

    off = pl.multiple_of(s * npt, 8)

    @pl.when(c == 0)
    def _():
        pltpu.sync_copy(accum.at[pl.ds(off, npt)],
                        nagga_hbm.at[pl.ds(off, npt)])

    @pl.when(c == 1)
    def _():
        pltpu.sync_copy(accum.at[pl.ds(off, npt)],
                        naggb_hbm.at[pl.ds(off, npt)])


def _scale_offset(acc, count, gamma, beta):
    mu = acc[0] / count
    var = acc[1] / count - mu * mu
    s = gamma * lax.rsqrt(var + BN_EPS)
    return s, beta - mu * s


def _blk(shape, imap):
    return pl.BlockSpec(shape, imap)


def kernel(x, e, edge_index, W1, b1, W2, b2, g2, be2, W3, b3, g3, be3,
           W4, b4, g4, be4, W5, b5, g5, be5):
    N, D = x.shape
    E = e.shape[0]
    H = W2.shape[1]
    f32 = jnp.float32
    j = edge_index[0]
    i = edge_index[1]
    ivr = i.reshape(E // _S2, _S2)

    W1a, W1b = W1[:D], W1[D:]
    z16 = jnp.zeros((16,), f32)
    Wt0 = jnp.concatenate([W1a[:, :16], W4[:, :16]], axis=1)
    Wt1 = jnp.concatenate([W1a[:, 16:], W4[:, 16:]], axis=1)
    bt0 = jnp.concatenate([b1[:16], z16]).reshape(1, 32)
    bt1 = jnp.concatenate([b1[16:], z16]).reshape(1, 32)
    Wc0 = jnp.concatenate([W1b[:, :16], W4[:, :16]], axis=1)
    Wc1 = jnp.concatenate([W1b[:, 16:], W4[:, 16:]], axis=1)
    bc0 = jnp.concatenate([z16, b4[:16]]).reshape(1, 32)
    bc1 = jnp.concatenate([z16, b4[16:]]).reshape(1, 32)

    row = lambda idx: (idx, 0)
    zero2 = lambda idx: (0, 0)
    first3 = lambda idx: (0, idx, 0)

    BN = 10000
    GN = N // BN
    TJ = pl.pallas_call(
        _tables_kernel,
        grid=(GN,),
        in_specs=[_blk((BN, D), row), _blk((D, 32), zero2), _blk((D, 32), zero2),
                  _blk((1, 32), zero2), _blk((1, 32), zero2)],
        out_specs=_blk((2, BN, _TW), first3),
        out_shape=jax.ShapeDtypeStruct((2, N, _TW), f32),
    )(x, Wt0, Wt1, bt0, bt1)

    BE = 8000
    GE = E // BE
    EW = pl.pallas_call(
        _ew_kernel,
        grid=(GE,),
        in_specs=[_blk((BE, D), row), _blk((D, 32), zero2), _blk((D, 32), zero2),
                  _blk((1, 32), zero2), _blk((1, 32), zero2)],
        out_specs=_blk((2, BE, 32), first3),
        out_shape=jax.ShapeDtypeStruct((2, E, 32), f32),
    )(e, Wc0, Wc1, bc0, bc1)

    tj2 = TJ.reshape(2 * N, _TW)
    ew2 = EW.reshape(2 * E, 32)

    Npad = 102400
    jc = jnp.concatenate([j, j + N])
    ic = jnp.concatenate([i, i + N])
    mesh = plsc.VectorSubcoreMesh(core_axis_name="c", subcore_axis_name="s",
                                  num_cores=_NC, num_subcores=_NS)
    sc1 = pl.kernel(
        functools.partial(_sc1_body, E),
        out_type=[jax.ShapeDtypeStruct((E, 16), f32),
                  jax.ShapeDtypeStruct((E, 16), f32),
                  jax.ShapeDtypeStruct((E, 16), f32),
                  jax.ShapeDtypeStruct((E, 16), f32),
                  jax.ShapeDtypeStruct((_NC * _NS, 2, 16), f32)],
        mesh=mesh,
        scratch_types=[
            pltpu.VMEM((2, _CH), jnp.int32),
            pltpu.VMEM((2, _CH), jnp.int32),
            pltpu.VMEM((2, _CH, _TW), f32),
            pltpu.VMEM((2, _CH, _TW), f32),
            pltpu.VMEM((2, _CH, 32), f32),
            pltpu.VMEM((2, _CH, 16), f32),
            pltpu.VMEM((2, _CH, 16), f32),
            pltpu.VMEM((2, 16), f32),
            pltpu.SemaphoreType.DMA,
            pltpu.SemaphoreType.DMA,
            pltpu.SemaphoreType.DMA,
            pltpu.SemaphoreType.DMA,
            pltpu.SemaphoreType.DMA,
            pltpu.SemaphoreType.DMA,
        ],
    )
    msga, msgb, z4a, z4b, scstats = sc1(tj2, ew2, jc, ic)

    sc2 = pl.kernel(
        functools.partial(_sc2_body, Npad, E),
        out_type=[jax.ShapeDtypeStruct((Npad, 16), f32),
                  jax.ShapeDtypeStruct((Npad, 16), f32)],
        mesh=mesh,
        scratch_types=[
            pltpu.VMEM_SHARED((Npad, 16), f32),
            pltpu.VMEM((2, _C2, 16), f32),
            pltpu.VMEM((2, _S2), jnp.int32),
            pltpu.SemaphoreType.DMA,
            pltpu.SemaphoreType.DMA,
        ],
    )
    _BISECT_SC2 = False
    if _BISECT_SC2:
        nagga, naggb = sc2(msga, msgb, ivr)
        nagga = nagga[:N]
        naggb = naggb[:N]
    else:
        del sc2, ivr
        nagga = jnp.zeros((N, 16), f32).at[i].add(msga)
        naggb = jnp.zeros((N, 16), f32).at[i].add(msgb)

    ssum = jnp.concatenate([scstats[:_NS, 0].sum(axis=0),
                            scstats[_NS:, 0].sum(axis=0)])
    ssq = jnp.concatenate([scstats[:_NS, 1].sum(axis=0),
                           scstats[_NS:, 1].sum(axis=0)])
    mu4 = ssum / E
    var4 = ssq / E - mu4 * mu4
    s4 = g4 * lax.rsqrt(var4 + BN_EPS)
    o4 = be4 - mu4 * s4

    z2, acc2 = pl.pallas_call(
        _node1_kernel,
        grid=(GN,),
        in_specs=[_blk((BN, 16), row), _blk((BN, 16), row), _blk((BN, D), row),
                  _blk((16, H), zero2), _blk((16, H), zero2), _blk((1, H), zero2)],
        out_specs=[_blk((BN, H), row), _blk((2, H), zero2)],
        out_shape=[jax.ShapeDtypeStruct((N, H), f32),
                   jax.ShapeDtypeStruct((2, H), f32)],
    )(nagga, naggb, x, W2[:16], W2[16:], b2.reshape(1, H))
    s2, o2 = _scale_offset(acc2, N, g2, be2)

    z3, acc3 = pl.pallas_call(
        _bn_lin_stats_kernel,
        grid=(GN,),
        in_specs=[_blk((BN, H), row), _blk((1, H), zero2), _blk((1, H), zero2),
                  _blk((H, H), zero2), _blk((1, H), zero2)],
        out_specs=[_blk((BN, H), row), _blk((2, H), zero2)],
        out_shape=[jax.ShapeDtypeStruct((N, H), f32),
                   jax.ShapeDtypeStruct((2, H), f32)],
    )(z2, s2.reshape(1, H), o2.reshape(1, H), W3, b3.reshape(1, H))
    s3, o3 = _scale_offset(acc3, N, g3, be3)

    n_out = pl.pallas_call(
        _bn_relu_kernel,
        grid=(GN,),
        in_specs=[_blk((BN, H), row), _blk((1, H), zero2), _blk((1, H), zero2)],
        out_specs=_blk((BN, H), row),
        out_shape=jax.ShapeDtypeStruct((N, H), f32),
    )(z3, s3.reshape(1, H), o3.reshape(1, H))

    acc5 = pl.pallas_call(
        _edge_stats_kernel,
        grid=(GE,),
        in_specs=[_blk((BE, 16), row), _blk((BE, 16), row),
                  _blk((1, H), zero2), _blk((1, H), zero2),
                  _blk((16, H), zero2), _blk((16, H), zero2), _blk((1, H), zero2)],
        out_specs=_blk((2, H), zero2),
        out_shape=jax.ShapeDtypeStruct((2, H), f32),
    )(z4a, z4b, s4.reshape(1, H), o4.reshape(1, H), W5[:16], W5[16:],
      b5.reshape(1, H))
    s5, o5 = _scale_offset(acc5, E, g5, be5)

    e_out = pl.pallas_call(
        _edge_out_kernel,
        grid=(GE,),
        in_specs=[_blk((BE, 16), row), _blk((BE, 16), row),
                  _blk((1, H), zero2), _blk((1, H), zero2),
                  _blk((16, H), zero2), _blk((16, H), zero2), _blk((1, H), zero2),
                  _blk((1, H), zero2), _blk((1, H), zero2)],
        out_specs=_blk((BE, H), row),
        out_shape=jax.ShapeDtypeStruct((E, H), f32),
    )(z4a, z4b, s4.reshape(1, H), o4.reshape(1, H), W5[:16], W5[16:],
      b5.reshape(1, H), s5.reshape(1, H), o5.reshape(1, H))

    return (n_out, e_out)
